# Initial kernel scaffold; baseline (speedup 1.0000x reference)
import functools

import jax
import jax.numpy as jnp
from jax import lax
from jax.experimental import pallas as pl
from jax.experimental.pallas import tpu as pltpu
from jax.experimental.pallas import tpu_sc as plsc

N = 10000
E = 320000
NPAD = 10240
RPT = 640
E_PAD = 323584
DC = 128
RB = 1000
NEG_SLOPE = 0.2
LN_EPS = 1e-5


def _make_segsum(num_chunks, edge_split):
  n_out = 2 if edge_split else num_chunks
  npass = 1 if edge_split else num_chunks // 2
  ept = E_PAD // 32 if edge_split else E_PAD // 16
  nblk = ept // 128
  mesh = plsc.VectorSubcoreMesh(core_axis_name="c", subcore_axis_name="s")
  out_type = [jax.ShapeDtypeStruct((NPAD, DC), jnp.float32)
              for _ in range(n_out)]
  scratch_types = [
      pltpu.VMEM((ept,), jnp.int32),
      pltpu.VMEM((nblk, 128), jnp.int32),
      pltpu.VMEM((128, DC), jnp.float32),
      pltpu.VMEM((128, DC), jnp.float32),
      pltpu.VMEM_SHARED((NPAD, DC), jnp.float32),
  ]

  def body(*refs):
    hs = refs[:num_chunks]
    src_hbm = refs[num_chunks]
    dst_hbm = refs[num_chunks + 1]
    outs = refs[num_chunks + 2:num_chunks + 2 + n_out]
    srcv, dstv, rows, zrows, acc = refs[num_chunks + 2 + n_out:]
    cid = lax.axis_index("c")
    sid = lax.axis_index("s")

    def _zero_row(i, carry):
      for j in range(DC // 16):
        zrows[i, pl.ds(j * 16, 16)] = jnp.zeros((16,), jnp.float32)
      return carry
    lax.fori_loop(0, 128, _zero_row, 0)

    wid = cid * 16 + sid if edge_split else sid
    pltpu.sync_copy(src_hbm.at[pl.ds(wid * ept, ept)], srcv)
    pltpu.sync_copy(dst_hbm.at[pl.ds(wid * nblk, nblk)], dstv)

    def run(h_ref):
      def blk(b, carry):
        pltpu.sync_copy(h_ref.at[srcv.at[pl.ds(b * 128, 128)]], rows)
        pltpu.sync_copy(rows, acc.at[dstv.at[b]], add=True)
        return carry
      lax.fori_loop(0, nblk, blk, 0)

    def dump(out_ref):
      for k in range(RPT // 128):
        r0 = sid * RPT + k * 128
        pltpu.sync_copy(acc.at[pl.ds(r0, 128)], out_ref.at[pl.ds(r0, 128)])

    for p in range(npass):
      for k in range(RPT // 128):
        pltpu.sync_copy(zrows, acc.at[pl.ds(sid * RPT + k * 128, 128)])
      plsc.subcore_barrier()
      if edge_split:
        run(hs[0])
      else:
        @pl.when(cid == 0)
        def _():
          run(hs[2 * p])

        @pl.when(cid == 1)
        def _():
          run(hs[2 * p + 1])
      plsc.subcore_barrier()
      o0, o1 = (outs[0], outs[1]) if edge_split else (outs[2 * p],
                                                      outs[2 * p + 1])

      @pl.when(cid == 0)
      def _():
        dump(o0)

      @pl.when(cid == 1)
      def _():
        dump(o1)
      if p + 1 < npass:
        plsc.subcore_barrier()

  return pl.kernel(body, out_type=out_type, mesh=mesh,
                   scratch_types=scratch_types)


def _ln_lrelu(z, g, b):
  mu = jnp.mean(z, axis=-1, keepdims=True)
  var = jnp.mean((z - mu) ** 2, axis=-1, keepdims=True)
  h = (z - mu) * lax.rsqrt(var + LN_EPS) * g + b
  return jnp.where(h >= 0, h, NEG_SLOPE * h)


def _full(shape):
  return pl.BlockSpec(shape, lambda i: (0, 0))


def _rows(width):
  return pl.BlockSpec((RB, width), lambda i: (i, 0))


def _rep8(v):
  return jnp.broadcast_to(v.reshape(1, -1), (8, v.shape[-1]))


def _k_layer0(x, p0, p1, wrel0, wroot0, b0, g0, bb0, wrel1):
  def body(x_r, p0_r, p1_r, wrel0_r, wroot0_r, b0_r, g0_r, bb0_r, wrel1_r,
           h1_r, y1_r):
    a = p0_r[...] + p1_r[...]
    z = jnp.dot(a, wrel0_r[...]) + jnp.dot(x_r[...], wroot0_r[...]) + b0_r[0:1]
    h1 = _ln_lrelu(z, g0_r[0:1], bb0_r[0:1])
    h1_r[...] = h1
    y1_r[...] = jnp.dot(h1, wrel1_r[...])

  return pl.pallas_call(
      body,
      grid=(N // RB,),
      in_specs=[
          _rows(128), _rows(128), _rows(128),
          _full((128, 64)), _full((128, 64)),
          _full((8, 64)), _full((8, 64)), _full((8, 64)),
          _full((64, 128)),
      ],
      out_specs=[_rows(64), _rows(128)],
      out_shape=[
          jax.ShapeDtypeStruct((N, 64), jnp.float32),
          jax.ShapeDtypeStruct((NPAD, 128), jnp.float32),
      ],
  )(x, p0, p1, wrel0, wroot0, _rep8(b0), _rep8(g0), _rep8(bb0), wrel1)


def _k_layer1(h1, p0, p1, wroot1, b1, g1, bb1):
  def body(h1_r, p0_r, p1_r, wroot_r, b_r, g_r, bb_r, h2_r):
    z = p0_r[...] + p1_r[...] + jnp.dot(h1_r[...], wroot_r[...]) + b_r[0:1]
    h2_r[...] = _ln_lrelu(z, g_r[0:1], bb_r[0:1])

  return pl.pallas_call(
      body,
      grid=(N // RB,),
      in_specs=[
          _rows(64), _rows(128), _rows(128),
          _full((64, 128)), _full((8, 128)), _full((8, 128)), _full((8, 128)),
      ],
      out_specs=[_rows(128)],
      out_shape=[jax.ShapeDtypeStruct((NPAD, 128), jnp.float32)],
  )(h1, p0, p1, wroot1, _rep8(b1), _rep8(g1), _rep8(bb1))


def _k_layer2(h2, p0, p1, wrel2, wroot2, b2, g2, bb2):
  def body(h2_r, p0_r, p1_r, wrel_r, wroot_r, b_r, g_r, bb_r, o0_r, o1_r):
    a = p0_r[...] + p1_r[...]
    z = jnp.dot(a, wrel_r[...]) + jnp.dot(h2_r[...], wroot_r[...]) + b_r[0:1]
    h3 = _ln_lrelu(z, g_r[0:1], bb_r[0:1])
    o0_r[...] = h3[:, :128]
    o1_r[...] = h3[:, 128:]

  return pl.pallas_call(
      body,
      grid=(N // RB,),
      in_specs=[
          _rows(128), _rows(128), _rows(128),
          _full((128, 256)), _full((128, 256)),
          _full((8, 256)), _full((8, 256)), _full((8, 256)),
      ],
      out_specs=[_rows(128), _rows(128)],
      out_shape=[jax.ShapeDtypeStruct((NPAD, 128), jnp.float32)] * 2,
  )(h2, p0, p1, wrel2, wroot2, _rep8(b2), _rep8(g2), _rep8(bb2))


def _k_layer3(h3s, a3s, wrel3, wroot3, b3, g3, bb3):
  def body(h30_r, h31_r, a30_r, a31_r, wrel_r, wroot_r, b_r, g_r, bb_r,
           *outs):
    z = (jnp.dot(a30_r[...], wrel_r[0:128]) +
         jnp.dot(a31_r[...], wrel_r[128:256]) +
         jnp.dot(h30_r[...], wroot_r[0:128]) +
         jnp.dot(h31_r[...], wroot_r[128:256]) + b_r[0:1])
    h4 = _ln_lrelu(z, g_r[0:1], bb_r[0:1])
    for c in range(4):
      outs[c][...] = h4[:, c * 128:(c + 1) * 128]

  return pl.pallas_call(
      body,
      grid=(N // RB,),
      in_specs=[
          _rows(128), _rows(128), _rows(128), _rows(128),
          _full((256, 512)), _full((256, 512)),
          _full((8, 512)), _full((8, 512)), _full((8, 512)),
      ],
      out_specs=[_rows(128)] * 4,
      out_shape=[jax.ShapeDtypeStruct((NPAD, 128), jnp.float32)] * 4,
  )(*h3s, *a3s, wrel3, wroot3, _rep8(b3), _rep8(g3), _rep8(bb3))


def _k_final(h4s, a4s, wrel4, wroot4, b4, g4, bb4, wfc, bfc):
  nblocks = N // RB

  def body(h40_r, h41_r, h42_r, h43_r, a40_r, a41_r, a42_r, a43_r,
           wrel_r, wroot_r, b_r, g_r, bb_r, wfc_r, bfc_r, out_r, acc):
    i = pl.program_id(0)
    hs = (h40_r, h41_r, h42_r, h43_r)
    as_ = (a40_r, a41_r, a42_r, a43_r)
    z = b_r[0:1] + jnp.zeros((RB, 512), jnp.float32)
    for c in range(4):
      z = z + jnp.dot(as_[c][...], wrel_r[pl.ds(c * 128, 128)])
      z = z + jnp.dot(hs[c][...], wroot_r[pl.ds(c * 128, 128)])
    h5 = _ln_lrelu(z, g_r[0:1], bb_r[0:1])
    part = jnp.sum(h5.reshape(RB // 8, 8, 512), axis=0)

    @pl.when(i == 0)
    def _():
      acc[...] = part

    @pl.when(i > 0)
    def _():
      acc[...] = acc[...] + part

    @pl.when(i == nblocks - 1)
    def _():
      total = jnp.sum(acc[...] * wfc_r[0:1]) + bfc_r[0, 0]
      out_r[...] = jnp.full((8, 128), total, jnp.float32)

  return pl.pallas_call(
      body,
      grid=(nblocks,),
      in_specs=[_rows(128)] * 8 + [
          _full((512, 512)), _full((512, 512)),
          _full((8, 512)), _full((8, 512)), _full((8, 512)),
          _full((8, 512)), _full((8, 128)),
      ],
      out_specs=[pl.BlockSpec((8, 128), lambda i: (0, 0))],
      out_shape=[jax.ShapeDtypeStruct((8, 128), jnp.float32)],
      scratch_shapes=[pltpu.VMEM((8, 512), jnp.float32)],
  )(*h4s, *a4s, wrel4, wroot4, _rep8(b4), _rep8(g4), _rep8(bb4),
    _rep8(wfc.reshape(-1)), jnp.broadcast_to(bfc.reshape(1, 1), (8, 128)))


def kernel(x, edge_index, params):
  src = edge_index[0].astype(jnp.int32)
  dst = edge_index[1].astype(jnp.int32)
  src_p = jnp.concatenate([src, jnp.zeros((E_PAD - E,), jnp.int32)])
  dst_p = jnp.concatenate([dst, jnp.full((E_PAD - E,), N, jnp.int32)])
  dst2d = dst_p.reshape(E_PAD // 128, 128)

  seg_edge = _make_segsum(1, edge_split=True)
  seg_feat2 = _make_segsum(2, edge_split=False)
  seg_feat4 = _make_segsum(4, edge_split=False)

  p = params
  q0, q1 = seg_edge(x, src_p, dst2d)
  h1, y1 = _k_layer0(x, q0, q1, p["W_rel0"], p["W_root0"], p["b0"],
                     p["ln_g0"], p["ln_b0"], p["W_rel1"])
  q0, q1 = seg_edge(y1, src_p, dst2d)
  h2 = _k_layer1(h1, q0, q1, p["W_root1"], p["b1"], p["ln_g1"], p["ln_b1"])[0]
  q0, q1 = seg_edge(h2, src_p, dst2d)
  h3s = _k_layer2(h2, q0, q1, p["W_rel2"], p["W_root2"], p["b2"],
                  p["ln_g2"], p["ln_b2"])
  a3s = seg_feat2(*h3s, src_p, dst2d)
  h4s = _k_layer3(h3s, a3s, p["W_rel3"], p["W_root3"], p["b3"],
                  p["ln_g3"], p["ln_b3"])
  a4s = seg_feat4(*h4s, src_p, dst2d)
  out = _k_final(h4s, a4s, p["W_rel4"], p["W_root4"], p["b4"],
                 p["ln_g4"], p["ln_b4"], p["W_fc"], p["b_fc"])[0]
  return out[0:1, 0]


# speedup vs baseline: 2.4351x; 2.4351x over previous
import functools

import jax
import jax.numpy as jnp
from jax import lax
from jax.experimental import pallas as pl
from jax.experimental.pallas import tpu as pltpu
from jax.experimental.pallas import tpu_sc as plsc

N = 10000
E = 320000
NPAD = 10240
RPT = 640
E_PAD = 327680
DC = 128
RB = 1000
NEG_SLOPE = 0.2
LN_EPS = 1e-5


def _make_segsum(num_chunks, edge_split):
  n_out = 2 if edge_split else num_chunks
  npass = 1 if edge_split else num_chunks // 2
  ept = E_PAD // 32 if edge_split else E_PAD // 16
  nblk = ept // 128
  mesh = plsc.VectorSubcoreMesh(core_axis_name="c", subcore_axis_name="s")
  out_type = [jax.ShapeDtypeStruct((NPAD, DC), jnp.float32)
              for _ in range(n_out)]
  scratch_types = [
      pltpu.VMEM((1024,), jnp.int32),
      pltpu.VMEM((8, 128), jnp.int32),
      pltpu.VMEM((128, DC), jnp.float32),
      pltpu.VMEM_SHARED((NPAD, DC), jnp.float32),
  ]

  def body(*refs):
    hs = refs[:num_chunks]
    src_hbm = refs[num_chunks]
    dst_hbm = refs[num_chunks + 1]
    outs = refs[num_chunks + 2:num_chunks + 2 + n_out]
    srcv, dstv, rows, acc = refs[num_chunks + 2 + n_out:]
    cid = lax.axis_index("c")
    sid = lax.axis_index("s")

    def _zero_rows():
      def _zr(i, carry):
        for j in range(DC // 16):
          rows[i, pl.ds(j * 16, 16)] = jnp.zeros((16,), jnp.float32)
        return carry
      lax.fori_loop(0, 128, _zr, 0)

    wid = cid * 16 + sid if edge_split else sid

    def run(h_ref):
      def sblk(g, carry):
        r0 = wid * nblk + g * 8
        pltpu.sync_copy(src_hbm.at[pl.ds(r0 * 128, 1024)], srcv)
        pltpu.sync_copy(dst_hbm.at[pl.ds(r0, 8)], dstv)
        for j in range(8):
          pltpu.sync_copy(h_ref.at[srcv.at[pl.ds(j * 128, 128)]], rows)
          pltpu.sync_copy(rows, acc.at[dstv.at[j]], add=True)
        return carry
      lax.fori_loop(0, nblk // 8, sblk, 0)

    def dump(out_ref):
      for k in range(RPT // 128):
        r0 = sid * RPT + k * 128
        pltpu.sync_copy(acc.at[pl.ds(r0, 128)], out_ref.at[pl.ds(r0, 128)])

    for p in range(npass):
      _zero_rows()
      for k in range(RPT // 128):
        pltpu.sync_copy(rows, acc.at[pl.ds(sid * RPT + k * 128, 128)])
      plsc.subcore_barrier()
      if edge_split:
        run(hs[0])
      else:
        @pl.when(cid == 0)
        def _():
          run(hs[2 * p])

        @pl.when(cid == 1)
        def _():
          run(hs[2 * p + 1])
      plsc.subcore_barrier()
      o0, o1 = (outs[0], outs[1]) if edge_split else (outs[2 * p],
                                                      outs[2 * p + 1])

      @pl.when(cid == 0)
      def _():
        dump(o0)

      @pl.when(cid == 1)
      def _():
        dump(o1)
      if p + 1 < npass:
        plsc.subcore_barrier()

  return pl.kernel(body, out_type=out_type, mesh=mesh,
                   scratch_types=scratch_types)


def _ln_lrelu(z, g, b):
  mu = jnp.mean(z, axis=-1, keepdims=True)
  var = jnp.mean((z - mu) ** 2, axis=-1, keepdims=True)
  h = (z - mu) * lax.rsqrt(var + LN_EPS) * g + b
  return jnp.where(h >= 0, h, NEG_SLOPE * h)


def _full(shape):
  return pl.BlockSpec(shape, lambda i: (0, 0))


def _rows(width):
  return pl.BlockSpec((RB, width), lambda i: (i, 0))


def _rep8(v):
  return jnp.broadcast_to(v.reshape(1, -1), (8, v.shape[-1]))


def _k_layer0(x, p0, p1, wrel0, wroot0, b0, g0, bb0, wrel1):
  def body(x_r, p0_r, p1_r, wrel0_r, wroot0_r, b0_r, g0_r, bb0_r, wrel1_r,
           h1_r, y1_r):
    a = p0_r[...] + p1_r[...]
    z = jnp.dot(a, wrel0_r[...]) + jnp.dot(x_r[...], wroot0_r[...]) + b0_r[0:1]
    h1 = _ln_lrelu(z, g0_r[0:1], bb0_r[0:1])
    h1_r[...] = h1
    y1_r[...] = jnp.dot(h1, wrel1_r[...])

  return pl.pallas_call(
      body,
      grid=(N // RB,),
      in_specs=[
          _rows(128), _rows(128), _rows(128),
          _full((128, 64)), _full((128, 64)),
          _full((8, 64)), _full((8, 64)), _full((8, 64)),
          _full((64, 128)),
      ],
      out_specs=[_rows(64), _rows(128)],
      out_shape=[
          jax.ShapeDtypeStruct((N, 64), jnp.float32),
          jax.ShapeDtypeStruct((NPAD, 128), jnp.float32),
      ],
  )(x, p0, p1, wrel0, wroot0, _rep8(b0), _rep8(g0), _rep8(bb0), wrel1)


def _k_layer1(h1, p0, p1, wroot1, b1, g1, bb1):
  def body(h1_r, p0_r, p1_r, wroot_r, b_r, g_r, bb_r, h2_r):
    z = p0_r[...] + p1_r[...] + jnp.dot(h1_r[...], wroot_r[...]) + b_r[0:1]
    h2_r[...] = _ln_lrelu(z, g_r[0:1], bb_r[0:1])

  return pl.pallas_call(
      body,
      grid=(N // RB,),
      in_specs=[
          _rows(64), _rows(128), _rows(128),
          _full((64, 128)), _full((8, 128)), _full((8, 128)), _full((8, 128)),
      ],
      out_specs=[_rows(128)],
      out_shape=[jax.ShapeDtypeStruct((NPAD, 128), jnp.float32)],
  )(h1, p0, p1, wroot1, _rep8(b1), _rep8(g1), _rep8(bb1))


def _k_layer2(h2, p0, p1, wrel2, wroot2, b2, g2, bb2):
  def body(h2_r, p0_r, p1_r, wrel_r, wroot_r, b_r, g_r, bb_r, o0_r, o1_r):
    a = p0_r[...] + p1_r[...]
    z = jnp.dot(a, wrel_r[...]) + jnp.dot(h2_r[...], wroot_r[...]) + b_r[0:1]
    h3 = _ln_lrelu(z, g_r[0:1], bb_r[0:1])
    o0_r[...] = h3[:, :128]
    o1_r[...] = h3[:, 128:]

  return pl.pallas_call(
      body,
      grid=(N // RB,),
      in_specs=[
          _rows(128), _rows(128), _rows(128),
          _full((128, 256)), _full((128, 256)),
          _full((8, 256)), _full((8, 256)), _full((8, 256)),
      ],
      out_specs=[_rows(128), _rows(128)],
      out_shape=[jax.ShapeDtypeStruct((NPAD, 128), jnp.float32)] * 2,
  )(h2, p0, p1, wrel2, wroot2, _rep8(b2), _rep8(g2), _rep8(bb2))


def _k_layer3(h3s, a3s, wrel3, wroot3, b3, g3, bb3):
  def body(h30_r, h31_r, a30_r, a31_r, wrel_r, wroot_r, b_r, g_r, bb_r,
           *outs):
    z = (jnp.dot(a30_r[...], wrel_r[0:128]) +
         jnp.dot(a31_r[...], wrel_r[128:256]) +
         jnp.dot(h30_r[...], wroot_r[0:128]) +
         jnp.dot(h31_r[...], wroot_r[128:256]) + b_r[0:1])
    h4 = _ln_lrelu(z, g_r[0:1], bb_r[0:1])
    for c in range(4):
      outs[c][...] = h4[:, c * 128:(c + 1) * 128]

  return pl.pallas_call(
      body,
      grid=(N // RB,),
      in_specs=[
          _rows(128), _rows(128), _rows(128), _rows(128),
          _full((256, 512)), _full((256, 512)),
          _full((8, 512)), _full((8, 512)), _full((8, 512)),
      ],
      out_specs=[_rows(128)] * 4,
      out_shape=[jax.ShapeDtypeStruct((NPAD, 128), jnp.float32)] * 4,
  )(*h3s, *a3s, wrel3, wroot3, _rep8(b3), _rep8(g3), _rep8(bb3))


def _k_final(h4s, a4s, wrel4, wroot4, b4, g4, bb4, wfc, bfc):
  nblocks = N // RB

  def body(h40_r, h41_r, h42_r, h43_r, a40_r, a41_r, a42_r, a43_r,
           wrel_r, wroot_r, b_r, g_r, bb_r, wfc_r, bfc_r, out_r, acc):
    i = pl.program_id(0)
    hs = (h40_r, h41_r, h42_r, h43_r)
    as_ = (a40_r, a41_r, a42_r, a43_r)
    z = b_r[0:1] + jnp.zeros((RB, 512), jnp.float32)
    for c in range(4):
      z = z + jnp.dot(as_[c][...], wrel_r[pl.ds(c * 128, 128)])
      z = z + jnp.dot(hs[c][...], wroot_r[pl.ds(c * 128, 128)])
    h5 = _ln_lrelu(z, g_r[0:1], bb_r[0:1])
    part = jnp.sum(h5.reshape(RB // 8, 8, 512), axis=0)

    @pl.when(i == 0)
    def _():
      acc[...] = part

    @pl.when(i > 0)
    def _():
      acc[...] = acc[...] + part

    @pl.when(i == nblocks - 1)
    def _():
      total = jnp.sum(acc[...] * wfc_r[0:1]) + bfc_r[0, 0]
      out_r[...] = jnp.full((8, 128), total, jnp.float32)

  return pl.pallas_call(
      body,
      grid=(nblocks,),
      in_specs=[_rows(128)] * 8 + [
          _full((512, 512)), _full((512, 512)),
          _full((8, 512)), _full((8, 512)), _full((8, 512)),
          _full((8, 512)), _full((8, 128)),
      ],
      out_specs=[pl.BlockSpec((8, 128), lambda i: (0, 0))],
      out_shape=[jax.ShapeDtypeStruct((8, 128), jnp.float32)],
      scratch_shapes=[pltpu.VMEM((8, 512), jnp.float32)],
  )(*h4s, *a4s, wrel4, wroot4, _rep8(b4), _rep8(g4), _rep8(bb4),
    _rep8(wfc.reshape(-1)), jnp.broadcast_to(bfc.reshape(1, 1), (8, 128)))


def kernel(x, edge_index, params):
  src = edge_index[0].astype(jnp.int32)
  dst = edge_index[1].astype(jnp.int32)
  src_p = jnp.concatenate([src, jnp.zeros((E_PAD - E,), jnp.int32)])
  dst_p = jnp.concatenate([dst, jnp.full((E_PAD - E,), N, jnp.int32)])
  dst2d = dst_p.reshape(E_PAD // 128, 128)

  seg_edge = _make_segsum(1, edge_split=True)
  seg_feat2 = _make_segsum(2, edge_split=False)
  seg_feat4 = _make_segsum(4, edge_split=False)

  p = params
  q0, q1 = seg_edge(x, src_p, dst2d)
  h1, y1 = _k_layer0(x, q0, q1, p["W_rel0"], p["W_root0"], p["b0"],
                     p["ln_g0"], p["ln_b0"], p["W_rel1"])
  q0, q1 = seg_edge(y1, src_p, dst2d)
  h2 = _k_layer1(h1, q0, q1, p["W_root1"], p["b1"], p["ln_g1"], p["ln_b1"])[0]
  q0, q1 = seg_edge(h2, src_p, dst2d)
  h3s = _k_layer2(h2, q0, q1, p["W_rel2"], p["W_root2"], p["b2"],
                  p["ln_g2"], p["ln_b2"])
  a3s = seg_feat2(*h3s, src_p, dst2d)
  h4s = _k_layer3(h3s, a3s, p["W_rel3"], p["W_root3"], p["b3"],
                  p["ln_g3"], p["ln_b3"])
  a4s = seg_feat4(*h4s, src_p, dst2d)
  out = _k_final(h4s, a4s, p["W_rel4"], p["W_root4"], p["b4"],
                 p["ln_g4"], p["ln_b4"], p["W_fc"], p["b_fc"])[0]
  return out[0:1, 0]


# speedup vs baseline: 6.9427x; 2.8511x over previous
import functools

import jax
import jax.numpy as jnp
from jax import lax
from jax.experimental import pallas as pl
from jax.experimental.pallas import tpu as pltpu
from jax.experimental.pallas import tpu_sc as plsc

N = 10000
E = 320000
NPAD = 10240
RPT = 640
E_PAD = 327680
DC = 128
RB = 1000
NEG_SLOPE = 0.2
LN_EPS = 1e-5


def _make_segsum(num_chunks, edge_split):
  n_out = 2 if edge_split else num_chunks
  npass = 1 if edge_split else num_chunks // 2
  ept = E_PAD // 32 if edge_split else E_PAD // 16
  nblk = ept // 128
  mesh = plsc.VectorSubcoreMesh(core_axis_name="c", subcore_axis_name="s")
  out_type = [jax.ShapeDtypeStruct((NPAD, DC), jnp.float32)
              for _ in range(n_out)]
  scratch_types = [
      pltpu.VMEM((1024,), jnp.int32),
      pltpu.VMEM((8, 128), jnp.int32),
      pltpu.VMEM((2, 128, DC), jnp.float32),
      pltpu.VMEM_SHARED((NPAD, DC), jnp.float32),
      pltpu.SemaphoreType.DMA,
      pltpu.SemaphoreType.DMA,
  ]

  def body(*refs):
    hs = refs[:num_chunks]
    src_hbm = refs[num_chunks]
    dst_hbm = refs[num_chunks + 1]
    outs = refs[num_chunks + 2:num_chunks + 2 + n_out]
    srcv, dstv, rows, acc, sg0, sg1 = refs[num_chunks + 2 + n_out:]
    sgs = (sg0, sg1)
    cid = lax.axis_index("c")
    sid = lax.axis_index("s")

    def _zero_rows():
      def _zr(i, carry):
        for j in range(DC // 16):
          rows[0, i, pl.ds(j * 16, 16)] = jnp.zeros((16,), jnp.float32)
        return carry
      lax.fori_loop(0, 128, _zr, 0)

    wid = cid * 16 + sid if edge_split else sid

    def run(h_ref):
      def sblk(g, carry):
        r0 = wid * nblk + g * 8
        pltpu.sync_copy(src_hbm.at[pl.ds(r0 * 128, 1024)], srcv)
        pltpu.sync_copy(dst_hbm.at[pl.ds(r0, 8)], dstv)

        def gath(j):
          return pltpu.async_copy(
              h_ref.at[srcv.at[pl.ds(j * 128, 128)]], rows.at[j % 2],
              sgs[j % 2])

        pend_g = gath(0)
        for j in range(8):
          pend_g.wait()
          if j < 7:
            pend_g = gath(j + 1)
          pltpu.sync_copy(rows.at[j % 2], acc.at[dstv.at[j]], add=True)
        return carry
      lax.fori_loop(0, nblk // 8, sblk, 0)

    def dump(out_ref):
      for k in range(RPT // 128):
        r0 = sid * RPT + k * 128
        pltpu.sync_copy(acc.at[pl.ds(r0, 128)], out_ref.at[pl.ds(r0, 128)])

    for p in range(npass):
      _zero_rows()
      for k in range(RPT // 128):
        pltpu.sync_copy(rows.at[0], acc.at[pl.ds(sid * RPT + k * 128, 128)])
      plsc.subcore_barrier()
      if edge_split:
        run(hs[0])
      else:
        @pl.when(cid == 0)
        def _():
          run(hs[2 * p])

        @pl.when(cid == 1)
        def _():
          run(hs[2 * p + 1])
      plsc.subcore_barrier()
      o0, o1 = (outs[0], outs[1]) if edge_split else (outs[2 * p],
                                                      outs[2 * p + 1])

      @pl.when(cid == 0)
      def _():
        dump(o0)

      @pl.when(cid == 1)
      def _():
        dump(o1)
      if p + 1 < npass:
        plsc.subcore_barrier()

  return pl.kernel(body, out_type=out_type, mesh=mesh,
                   scratch_types=scratch_types)


def _ln_lrelu(z, g, b):
  mu = jnp.mean(z, axis=-1, keepdims=True)
  var = jnp.mean((z - mu) ** 2, axis=-1, keepdims=True)
  h = (z - mu) * lax.rsqrt(var + LN_EPS) * g + b
  return jnp.where(h >= 0, h, NEG_SLOPE * h)


def _full(shape):
  return pl.BlockSpec(shape, lambda i: (0, 0))


def _rows(width):
  return pl.BlockSpec((RB, width), lambda i: (i, 0))


def _rep8(v):
  return jnp.broadcast_to(v.reshape(1, -1), (8, v.shape[-1]))


def _k_layer0(x, p0, p1, wrel0, wroot0, b0, g0, bb0, wrel1):
  def body(x_r, p0_r, p1_r, wrel0_r, wroot0_r, b0_r, g0_r, bb0_r, wrel1_r,
           h1_r, y1_r):
    a = p0_r[...] + p1_r[...]
    z = jnp.dot(a, wrel0_r[...]) + jnp.dot(x_r[...], wroot0_r[...]) + b0_r[0:1]
    h1 = _ln_lrelu(z, g0_r[0:1], bb0_r[0:1])
    h1_r[...] = h1
    y1_r[...] = jnp.dot(h1, wrel1_r[...])

  return pl.pallas_call(
      body,
      grid=(N // RB,),
      in_specs=[
          _rows(128), _rows(128), _rows(128),
          _full((128, 64)), _full((128, 64)),
          _full((8, 64)), _full((8, 64)), _full((8, 64)),
          _full((64, 128)),
      ],
      out_specs=[_rows(64), _rows(128)],
      out_shape=[
          jax.ShapeDtypeStruct((N, 64), jnp.float32),
          jax.ShapeDtypeStruct((NPAD, 128), jnp.float32),
      ],
  )(x, p0, p1, wrel0, wroot0, _rep8(b0), _rep8(g0), _rep8(bb0), wrel1)


def _k_layer1(h1, p0, p1, wroot1, b1, g1, bb1):
  def body(h1_r, p0_r, p1_r, wroot_r, b_r, g_r, bb_r, h2_r):
    z = p0_r[...] + p1_r[...] + jnp.dot(h1_r[...], wroot_r[...]) + b_r[0:1]
    h2_r[...] = _ln_lrelu(z, g_r[0:1], bb_r[0:1])

  return pl.pallas_call(
      body,
      grid=(N // RB,),
      in_specs=[
          _rows(64), _rows(128), _rows(128),
          _full((64, 128)), _full((8, 128)), _full((8, 128)), _full((8, 128)),
      ],
      out_specs=[_rows(128)],
      out_shape=[jax.ShapeDtypeStruct((NPAD, 128), jnp.float32)],
  )(h1, p0, p1, wroot1, _rep8(b1), _rep8(g1), _rep8(bb1))


def _k_layer2(h2, p0, p1, wrel2, wroot2, b2, g2, bb2):
  def body(h2_r, p0_r, p1_r, wrel_r, wroot_r, b_r, g_r, bb_r, o0_r, o1_r):
    a = p0_r[...] + p1_r[...]
    z = jnp.dot(a, wrel_r[...]) + jnp.dot(h2_r[...], wroot_r[...]) + b_r[0:1]
    h3 = _ln_lrelu(z, g_r[0:1], bb_r[0:1])
    o0_r[...] = h3[:, :128]
    o1_r[...] = h3[:, 128:]

  return pl.pallas_call(
      body,
      grid=(N // RB,),
      in_specs=[
          _rows(128), _rows(128), _rows(128),
          _full((128, 256)), _full((128, 256)),
          _full((8, 256)), _full((8, 256)), _full((8, 256)),
      ],
      out_specs=[_rows(128), _rows(128)],
      out_shape=[jax.ShapeDtypeStruct((NPAD, 128), jnp.float32)] * 2,
  )(h2, p0, p1, wrel2, wroot2, _rep8(b2), _rep8(g2), _rep8(bb2))


def _k_layer3(h3s, a3s, wrel3, wroot3, b3, g3, bb3):
  def body(h30_r, h31_r, a30_r, a31_r, wrel_r, wroot_r, b_r, g_r, bb_r,
           *outs):
    z = (jnp.dot(a30_r[...], wrel_r[0:128]) +
         jnp.dot(a31_r[...], wrel_r[128:256]) +
         jnp.dot(h30_r[...], wroot_r[0:128]) +
         jnp.dot(h31_r[...], wroot_r[128:256]) + b_r[0:1])
    h4 = _ln_lrelu(z, g_r[0:1], bb_r[0:1])
    for c in range(4):
      outs[c][...] = h4[:, c * 128:(c + 1) * 128]

  return pl.pallas_call(
      body,
      grid=(N // RB,),
      in_specs=[
          _rows(128), _rows(128), _rows(128), _rows(128),
          _full((256, 512)), _full((256, 512)),
          _full((8, 512)), _full((8, 512)), _full((8, 512)),
      ],
      out_specs=[_rows(128)] * 4,
      out_shape=[jax.ShapeDtypeStruct((NPAD, 128), jnp.float32)] * 4,
  )(*h3s, *a3s, wrel3, wroot3, _rep8(b3), _rep8(g3), _rep8(bb3))


def _k_final(h4s, a4s, wrel4, wroot4, b4, g4, bb4, wfc, bfc):
  nblocks = N // RB

  def body(h40_r, h41_r, h42_r, h43_r, a40_r, a41_r, a42_r, a43_r,
           wrel_r, wroot_r, b_r, g_r, bb_r, wfc_r, bfc_r, out_r, acc):
    i = pl.program_id(0)
    hs = (h40_r, h41_r, h42_r, h43_r)
    as_ = (a40_r, a41_r, a42_r, a43_r)
    z = b_r[0:1] + jnp.zeros((RB, 512), jnp.float32)
    for c in range(4):
      z = z + jnp.dot(as_[c][...], wrel_r[pl.ds(c * 128, 128)])
      z = z + jnp.dot(hs[c][...], wroot_r[pl.ds(c * 128, 128)])
    h5 = _ln_lrelu(z, g_r[0:1], bb_r[0:1])
    part = jnp.sum(h5.reshape(RB // 8, 8, 512), axis=0)

    @pl.when(i == 0)
    def _():
      acc[...] = part

    @pl.when(i > 0)
    def _():
      acc[...] = acc[...] + part

    @pl.when(i == nblocks - 1)
    def _():
      total = jnp.sum(acc[...] * wfc_r[0:1]) + bfc_r[0, 0]
      out_r[...] = jnp.full((8, 128), total, jnp.float32)

  return pl.pallas_call(
      body,
      grid=(nblocks,),
      in_specs=[_rows(128)] * 8 + [
          _full((512, 512)), _full((512, 512)),
          _full((8, 512)), _full((8, 512)), _full((8, 512)),
          _full((8, 512)), _full((8, 128)),
      ],
      out_specs=[pl.BlockSpec((8, 128), lambda i: (0, 0))],
      out_shape=[jax.ShapeDtypeStruct((8, 128), jnp.float32)],
      scratch_shapes=[pltpu.VMEM((8, 512), jnp.float32)],
  )(*h4s, *a4s, wrel4, wroot4, _rep8(b4), _rep8(g4), _rep8(bb4),
    _rep8(wfc.reshape(-1)), jnp.broadcast_to(bfc.reshape(1, 1), (8, 128)))


def kernel(x, edge_index, params):
  src = edge_index[0].astype(jnp.int32)
  dst = edge_index[1].astype(jnp.int32)
  npad_e = E_PAD - E
  pad_src = (jnp.arange(npad_e, dtype=jnp.int32) * 61) % N
  pad_dst = N + (jnp.arange(npad_e, dtype=jnp.int32) % (NPAD - N))
  src_p = jnp.concatenate([src, pad_src])
  dst_p = jnp.concatenate([dst, pad_dst])
  dst2d = dst_p.reshape(E_PAD // 128, 128)

  seg_edge = _make_segsum(1, edge_split=True)
  seg_feat2 = _make_segsum(2, edge_split=False)
  seg_feat4 = _make_segsum(4, edge_split=False)

  p = params
  q0, q1 = seg_edge(x, src_p, dst2d)
  h1, y1 = _k_layer0(x, q0, q1, p["W_rel0"], p["W_root0"], p["b0"],
                     p["ln_g0"], p["ln_b0"], p["W_rel1"])
  q0, q1 = seg_edge(y1, src_p, dst2d)
  h2 = _k_layer1(h1, q0, q1, p["W_root1"], p["b1"], p["ln_g1"], p["ln_b1"])[0]
  q0, q1 = seg_edge(h2, src_p, dst2d)
  h3s = _k_layer2(h2, q0, q1, p["W_rel2"], p["W_root2"], p["b2"],
                  p["ln_g2"], p["ln_b2"])
  a3s = seg_feat2(*h3s, src_p, dst2d)
  h4s = _k_layer3(h3s, a3s, p["W_rel3"], p["W_root3"], p["b3"],
                  p["ln_g3"], p["ln_b3"])
  a4s = seg_feat4(*h4s, src_p, dst2d)
  out = _k_final(h4s, a4s, p["W_rel4"], p["W_root4"], p["b4"],
                 p["ln_g4"], p["ln_b4"], p["W_fc"], p["b_fc"])[0]
  return out[0:1, 0]


# speedup vs baseline: 6.9446x; 1.0003x over previous
import functools

import jax
import jax.numpy as jnp
from jax import lax
from jax.experimental import pallas as pl
from jax.experimental.pallas import tpu as pltpu
from jax.experimental.pallas import tpu_sc as plsc

N = 10000
E = 320000
NPAD = 10240
RPT = 640
E_PAD = 327680
DC = 128
RB = 1000
NEG_SLOPE = 0.2
LN_EPS = 1e-5


def _make_segsum(num_chunks, edge_split):
  n_out = 2 if edge_split else num_chunks
  npass = 1 if edge_split else num_chunks // 2
  ept = E_PAD // 32 if edge_split else E_PAD // 16
  nblk = ept // 128
  mesh = plsc.VectorSubcoreMesh(core_axis_name="c", subcore_axis_name="s")
  out_type = [jax.ShapeDtypeStruct((NPAD, DC), jnp.float32)
              for _ in range(n_out)]
  scratch_types = [
      pltpu.VMEM((1024,), jnp.int32),
      pltpu.VMEM((8, 128), jnp.int32),
      pltpu.VMEM((2, 128, DC), jnp.float32),
      pltpu.VMEM_SHARED((NPAD, DC), jnp.float32),
      pltpu.SemaphoreType.DMA,
      pltpu.SemaphoreType.DMA,
      pltpu.SemaphoreType.DMA,
  ]

  def body(*refs):
    hs = refs[:num_chunks]
    src_hbm = refs[num_chunks]
    dst_hbm = refs[num_chunks + 1]
    outs = refs[num_chunks + 2:num_chunks + 2 + n_out]
    srcv, dstv, rows, acc, sg0, sg1, ssc = refs[num_chunks + 2 + n_out:]
    sgs = (sg0, sg1)
    cid = lax.axis_index("c")
    sid = lax.axis_index("s")

    def _zero_rows():
      def _zr(i, carry):
        for j in range(DC // 16):
          rows[0, i, pl.ds(j * 16, 16)] = jnp.zeros((16,), jnp.float32)
        return carry
      lax.fori_loop(0, 128, _zr, 0)

    wid = cid * 16 + sid if edge_split else sid

    def run(h_ref):
      def sblk(g, carry):
        r0 = wid * nblk + g * 8
        pltpu.sync_copy(src_hbm.at[pl.ds(r0 * 128, 1024)], srcv)
        pltpu.sync_copy(dst_hbm.at[pl.ds(r0, 8)], dstv)

        def gath(j):
          return pltpu.async_copy(
              h_ref.at[srcv.at[pl.ds(j * 128, 128)]], rows.at[j % 2],
              sgs[j % 2])

        def scat(j):
          return pltpu.async_copy(rows.at[j % 2], acc.at[dstv.at[j]], ssc,
                                  add=True)

        pend_g = gath(0)
        pend_s = None
        for j in range(8):
          pend_g.wait()
          if pend_s is not None:
            pend_s.wait()
          if j < 7:
            pend_g = gath(j + 1)
          pend_s = scat(j)
        pend_s.wait()
        return carry
      lax.fori_loop(0, nblk // 8, sblk, 0)

    def dump(out_ref):
      for k in range(RPT // 128):
        r0 = sid * RPT + k * 128
        pltpu.sync_copy(acc.at[pl.ds(r0, 128)], out_ref.at[pl.ds(r0, 128)])

    for p in range(npass):
      _zero_rows()
      for k in range(RPT // 128):
        pltpu.sync_copy(rows.at[0], acc.at[pl.ds(sid * RPT + k * 128, 128)])
      plsc.subcore_barrier()
      if edge_split:
        run(hs[0])
      else:
        @pl.when(cid == 0)
        def _():
          run(hs[2 * p])

        @pl.when(cid == 1)
        def _():
          run(hs[2 * p + 1])
      plsc.subcore_barrier()
      o0, o1 = (outs[0], outs[1]) if edge_split else (outs[2 * p],
                                                      outs[2 * p + 1])

      @pl.when(cid == 0)
      def _():
        dump(o0)

      @pl.when(cid == 1)
      def _():
        dump(o1)
      if p + 1 < npass:
        plsc.subcore_barrier()

  return pl.kernel(body, out_type=out_type, mesh=mesh,
                   scratch_types=scratch_types)


def _ln_lrelu(z, g, b):
  mu = jnp.mean(z, axis=-1, keepdims=True)
  var = jnp.mean((z - mu) ** 2, axis=-1, keepdims=True)
  h = (z - mu) * lax.rsqrt(var + LN_EPS) * g + b
  return jnp.where(h >= 0, h, NEG_SLOPE * h)


def _full(shape):
  return pl.BlockSpec(shape, lambda i: (0, 0))


def _rows(width):
  return pl.BlockSpec((RB, width), lambda i: (i, 0))


def _rep8(v):
  return jnp.broadcast_to(v.reshape(1, -1), (8, v.shape[-1]))


def _k_layer0(x, p0, p1, wrel0, wroot0, b0, g0, bb0, wrel1):
  def body(x_r, p0_r, p1_r, wrel0_r, wroot0_r, b0_r, g0_r, bb0_r, wrel1_r,
           h1_r, y1_r):
    a = p0_r[...] + p1_r[...]
    z = jnp.dot(a, wrel0_r[...]) + jnp.dot(x_r[...], wroot0_r[...]) + b0_r[0:1]
    h1 = _ln_lrelu(z, g0_r[0:1], bb0_r[0:1])
    h1_r[...] = h1
    y1_r[...] = jnp.dot(h1, wrel1_r[...])

  return pl.pallas_call(
      body,
      grid=(N // RB,),
      in_specs=[
          _rows(128), _rows(128), _rows(128),
          _full((128, 64)), _full((128, 64)),
          _full((8, 64)), _full((8, 64)), _full((8, 64)),
          _full((64, 128)),
      ],
      out_specs=[_rows(64), _rows(128)],
      out_shape=[
          jax.ShapeDtypeStruct((N, 64), jnp.float32),
          jax.ShapeDtypeStruct((NPAD, 128), jnp.float32),
      ],
  )(x, p0, p1, wrel0, wroot0, _rep8(b0), _rep8(g0), _rep8(bb0), wrel1)


def _k_layer1(h1, p0, p1, wroot1, b1, g1, bb1):
  def body(h1_r, p0_r, p1_r, wroot_r, b_r, g_r, bb_r, h2_r):
    z = p0_r[...] + p1_r[...] + jnp.dot(h1_r[...], wroot_r[...]) + b_r[0:1]
    h2_r[...] = _ln_lrelu(z, g_r[0:1], bb_r[0:1])

  return pl.pallas_call(
      body,
      grid=(N // RB,),
      in_specs=[
          _rows(64), _rows(128), _rows(128),
          _full((64, 128)), _full((8, 128)), _full((8, 128)), _full((8, 128)),
      ],
      out_specs=[_rows(128)],
      out_shape=[jax.ShapeDtypeStruct((NPAD, 128), jnp.float32)],
  )(h1, p0, p1, wroot1, _rep8(b1), _rep8(g1), _rep8(bb1))


def _k_layer2(h2, p0, p1, wrel2, wroot2, b2, g2, bb2):
  def body(h2_r, p0_r, p1_r, wrel_r, wroot_r, b_r, g_r, bb_r, o0_r, o1_r):
    a = p0_r[...] + p1_r[...]
    z = jnp.dot(a, wrel_r[...]) + jnp.dot(h2_r[...], wroot_r[...]) + b_r[0:1]
    h3 = _ln_lrelu(z, g_r[0:1], bb_r[0:1])
    o0_r[...] = h3[:, :128]
    o1_r[...] = h3[:, 128:]

  return pl.pallas_call(
      body,
      grid=(N // RB,),
      in_specs=[
          _rows(128), _rows(128), _rows(128),
          _full((128, 256)), _full((128, 256)),
          _full((8, 256)), _full((8, 256)), _full((8, 256)),
      ],
      out_specs=[_rows(128), _rows(128)],
      out_shape=[jax.ShapeDtypeStruct((NPAD, 128), jnp.float32)] * 2,
  )(h2, p0, p1, wrel2, wroot2, _rep8(b2), _rep8(g2), _rep8(bb2))


def _k_layer3(h3s, a3s, wrel3, wroot3, b3, g3, bb3):
  def body(h30_r, h31_r, a30_r, a31_r, wrel_r, wroot_r, b_r, g_r, bb_r,
           *outs):
    z = (jnp.dot(a30_r[...], wrel_r[0:128]) +
         jnp.dot(a31_r[...], wrel_r[128:256]) +
         jnp.dot(h30_r[...], wroot_r[0:128]) +
         jnp.dot(h31_r[...], wroot_r[128:256]) + b_r[0:1])
    h4 = _ln_lrelu(z, g_r[0:1], bb_r[0:1])
    for c in range(4):
      outs[c][...] = h4[:, c * 128:(c + 1) * 128]

  return pl.pallas_call(
      body,
      grid=(N // RB,),
      in_specs=[
          _rows(128), _rows(128), _rows(128), _rows(128),
          _full((256, 512)), _full((256, 512)),
          _full((8, 512)), _full((8, 512)), _full((8, 512)),
      ],
      out_specs=[_rows(128)] * 4,
      out_shape=[jax.ShapeDtypeStruct((NPAD, 128), jnp.float32)] * 4,
  )(*h3s, *a3s, wrel3, wroot3, _rep8(b3), _rep8(g3), _rep8(bb3))


def _k_final(h4s, a4s, wrel4, wroot4, b4, g4, bb4, wfc, bfc):
  nblocks = N // RB

  def body(h40_r, h41_r, h42_r, h43_r, a40_r, a41_r, a42_r, a43_r,
           wrel_r, wroot_r, b_r, g_r, bb_r, wfc_r, bfc_r, out_r, acc):
    i = pl.program_id(0)
    hs = (h40_r, h41_r, h42_r, h43_r)
    as_ = (a40_r, a41_r, a42_r, a43_r)
    z = b_r[0:1] + jnp.zeros((RB, 512), jnp.float32)
    for c in range(4):
      z = z + jnp.dot(as_[c][...], wrel_r[pl.ds(c * 128, 128)])
      z = z + jnp.dot(hs[c][...], wroot_r[pl.ds(c * 128, 128)])
    h5 = _ln_lrelu(z, g_r[0:1], bb_r[0:1])
    part = jnp.sum(h5.reshape(RB // 8, 8, 512), axis=0)

    @pl.when(i == 0)
    def _():
      acc[...] = part

    @pl.when(i > 0)
    def _():
      acc[...] = acc[...] + part

    @pl.when(i == nblocks - 1)
    def _():
      total = jnp.sum(acc[...] * wfc_r[0:1]) + bfc_r[0, 0]
      out_r[...] = jnp.full((8, 128), total, jnp.float32)

  return pl.pallas_call(
      body,
      grid=(nblocks,),
      in_specs=[_rows(128)] * 8 + [
          _full((512, 512)), _full((512, 512)),
          _full((8, 512)), _full((8, 512)), _full((8, 512)),
          _full((8, 512)), _full((8, 128)),
      ],
      out_specs=[pl.BlockSpec((8, 128), lambda i: (0, 0))],
      out_shape=[jax.ShapeDtypeStruct((8, 128), jnp.float32)],
      scratch_shapes=[pltpu.VMEM((8, 512), jnp.float32)],
  )(*h4s, *a4s, wrel4, wroot4, _rep8(b4), _rep8(g4), _rep8(bb4),
    _rep8(wfc.reshape(-1)), jnp.broadcast_to(bfc.reshape(1, 1), (8, 128)))


def kernel(x, edge_index, params):
  src = edge_index[0].astype(jnp.int32)
  dst = edge_index[1].astype(jnp.int32)
  npad_e = E_PAD - E
  pad_src = (jnp.arange(npad_e, dtype=jnp.int32) * 61) % N
  pad_dst = N + (jnp.arange(npad_e, dtype=jnp.int32) % (NPAD - N))
  src_p = jnp.concatenate([src, pad_src])
  dst_p = jnp.concatenate([dst, pad_dst])
  dst2d = dst_p.reshape(E_PAD // 128, 128)

  seg_edge = _make_segsum(1, edge_split=True)
  seg_feat2 = _make_segsum(2, edge_split=False)
  seg_feat4 = _make_segsum(4, edge_split=False)

  p = params
  q0, q1 = seg_edge(x, src_p, dst2d)
  h1, y1 = _k_layer0(x, q0, q1, p["W_rel0"], p["W_root0"], p["b0"],
                     p["ln_g0"], p["ln_b0"], p["W_rel1"])
  q0, q1 = seg_edge(y1, src_p, dst2d)
  h2 = _k_layer1(h1, q0, q1, p["W_root1"], p["b1"], p["ln_g1"], p["ln_b1"])[0]
  q0, q1 = seg_edge(h2, src_p, dst2d)
  h3s = _k_layer2(h2, q0, q1, p["W_rel2"], p["W_root2"], p["b2"],
                  p["ln_g2"], p["ln_b2"])
  a3s = seg_feat2(*h3s, src_p, dst2d)
  h4s = _k_layer3(h3s, a3s, p["W_rel3"], p["W_root3"], p["b3"],
                  p["ln_g3"], p["ln_b3"])
  a4s = seg_feat4(*h4s, src_p, dst2d)
  out = _k_final(h4s, a4s, p["W_rel4"], p["W_root4"], p["b4"],
                 p["ln_g4"], p["ln_b4"], p["W_fc"], p["b_fc"])[0]
  return out[0:1, 0]


# speedup vs baseline: 7.7457x; 1.1154x over previous
import functools

import jax
import jax.numpy as jnp
from jax import lax
from jax.experimental import pallas as pl
from jax.experimental.pallas import tpu as pltpu
from jax.experimental.pallas import tpu_sc as plsc

N = 10000
E = 320000
NPAD = 10240
RPT = 640
E_PAD = 327680
DC = 128
RB = 1000
NEG_SLOPE = 0.2
LN_EPS = 1e-5


def _make_segsum(num_chunks, edge_split, dc):
  n_out = 2 if edge_split else num_chunks
  npass = 1 if edge_split else num_chunks // 2
  ept = E_PAD // 32 if edge_split else E_PAD // 16
  nblk = ept // 128
  mesh = plsc.VectorSubcoreMesh(core_axis_name="c", subcore_axis_name="s")
  out_type = [jax.ShapeDtypeStruct((NPAD, dc), jnp.float32)
              for _ in range(n_out)]
  scratch_types = [
      pltpu.VMEM((2, 1024), jnp.int32),
      pltpu.VMEM((2, 8, 128), jnp.int32),
      pltpu.VMEM((2, 128, dc), jnp.float32),
      pltpu.VMEM_SHARED((NPAD, dc), jnp.float32),
      pltpu.SemaphoreType.DMA,
      pltpu.SemaphoreType.DMA,
      pltpu.SemaphoreType.DMA,
      pltpu.SemaphoreType.DMA,
      pltpu.SemaphoreType.DMA,
  ]

  def body(*refs):
    hs = refs[:num_chunks]
    src_hbm = refs[num_chunks]
    dst_hbm = refs[num_chunks + 1]
    outs = refs[num_chunks + 2:num_chunks + 2 + n_out]
    srcv, dstv, rows, acc, sg0, sg1, ssc, si0, si1 = \
        refs[num_chunks + 2 + n_out:]
    sgs = (sg0, sg1)
    cid = lax.axis_index("c")
    sid = lax.axis_index("s")

    def _zero_rows():
      def _zr(i, carry):
        for j in range(dc // 16):
          rows[0, i, pl.ds(j * 16, 16)] = jnp.zeros((16,), jnp.float32)
        return carry
      lax.fori_loop(0, 128, _zr, 0)

    wid = cid * 16 + sid if edge_split else sid
    nsup = nblk // 8

    def run(h_ref):
      def fire_idx(g, slot):
        r0 = wid * nblk + g * 8
        pltpu.async_copy(src_hbm.at[pl.ds(r0 * 128, 1024)], srcv.at[slot],
                         si0)
        pltpu.async_copy(dst_hbm.at[pl.ds(r0, 8)], dstv.at[slot], si1)

      def wait_idx():
        pltpu.make_async_copy(src_hbm.at[pl.ds(0, 1024)], srcv.at[0],
                              si0).wait()
        pltpu.make_async_copy(dst_hbm.at[pl.ds(0, 8)], dstv.at[0],
                              si1).wait()

      fire_idx(0, 0)

      def sblk(g, carry):
        slot = lax.rem(g, 2)
        wait_idx()

        @pl.when(g + 1 < nsup)
        def _():
          fire_idx(g + 1, 1 - slot)

        def gath(j):
          return pltpu.async_copy(
              h_ref.at[srcv.at[slot].at[pl.ds(j * 128, 128)]],
              rows.at[j % 2], sgs[j % 2])

        def scat(j):
          return pltpu.async_copy(rows.at[j % 2], acc.at[dstv.at[slot, j]],
                                  ssc, add=True)

        pend_g = gath(0)
        pend_s = None
        for j in range(8):
          pend_g.wait()
          if pend_s is not None:
            pend_s.wait()
          if j < 7:
            pend_g = gath(j + 1)
          pend_s = scat(j)
        pend_s.wait()
        return carry
      lax.fori_loop(0, nsup, sblk, 0)

    def dump(out_ref):
      for k in range(RPT // 128):
        r0 = sid * RPT + k * 128
        pltpu.sync_copy(acc.at[pl.ds(r0, 128)], out_ref.at[pl.ds(r0, 128)])

    for p in range(npass):
      _zero_rows()
      for k in range(RPT // 128):
        pltpu.sync_copy(rows.at[0], acc.at[pl.ds(sid * RPT + k * 128, 128)])
      plsc.subcore_barrier()
      if edge_split:
        run(hs[0])
      else:
        @pl.when(cid == 0)
        def _():
          run(hs[2 * p])

        @pl.when(cid == 1)
        def _():
          run(hs[2 * p + 1])
      plsc.subcore_barrier()
      o0, o1 = (outs[0], outs[1]) if edge_split else (outs[2 * p],
                                                      outs[2 * p + 1])

      @pl.when(cid == 0)
      def _():
        dump(o0)

      @pl.when(cid == 1)
      def _():
        dump(o1)
      if p + 1 < npass:
        plsc.subcore_barrier()

  cparams = None
  if dc % 128 != 0:
    cparams = pltpu.CompilerParams(use_tc_tiling_on_sc=False)
  return pl.kernel(body, out_type=out_type, mesh=mesh,
                   scratch_types=scratch_types, compiler_params=cparams)


def _ln_lrelu(z, g, b):
  mu = jnp.mean(z, axis=-1, keepdims=True)
  var = jnp.mean((z - mu) ** 2, axis=-1, keepdims=True)
  h = (z - mu) * lax.rsqrt(var + LN_EPS) * g + b
  return jnp.where(h >= 0, h, NEG_SLOPE * h)


def _full(shape):
  return pl.BlockSpec(shape, lambda i: (0, 0))


def _rows(width):
  return pl.BlockSpec((RB, width), lambda i: (i, 0))


def _rep8(v):
  return jnp.broadcast_to(v.reshape(1, -1), (8, v.shape[-1]))


def _k_pre(x, wrel0, wroot0, b0):
  def body(x_r, wrel_r, wroot_r, b_r, y0_r, r0_r):
    xv = x_r[...]
    y0_r[...] = jnp.dot(xv, wrel_r[...])
    r0_r[...] = jnp.dot(xv, wroot_r[...]) + b_r[0:1]

  return pl.pallas_call(
      body,
      grid=(N // RB,),
      in_specs=[
          _rows(128), _full((128, 64)), _full((128, 64)), _full((8, 64)),
      ],
      out_specs=[_rows(64), _rows(64)],
      out_shape=[
          jax.ShapeDtypeStruct((NPAD, 64), jnp.float32),
          jax.ShapeDtypeStruct((N, 64), jnp.float32),
      ],
  )(x, wrel0, wroot0, _rep8(b0))


def _k_layer0(r0, p0, p1, g0, bb0):
  def body(r0_r, p0_r, p1_r, g0_r, bb0_r, h1_r):
    z = p0_r[...] + p1_r[...] + r0_r[...]
    h1_r[...] = _ln_lrelu(z, g0_r[0:1], bb0_r[0:1])

  return pl.pallas_call(
      body,
      grid=(N // RB,),
      in_specs=[
          _rows(64), _rows(64), _rows(64), _full((8, 64)), _full((8, 64)),
      ],
      out_specs=[_rows(64)],
      out_shape=[jax.ShapeDtypeStruct((NPAD, 64), jnp.float32)],
  )(r0, p0, p1, _rep8(g0), _rep8(bb0))


def _k_layer1(h1, p0, p1, wrel1, wroot1, b1, g1, bb1):
  def body(h1_r, p0_r, p1_r, wrel_r, wroot_r, b_r, g_r, bb_r, h2_r):
    a = p0_r[...] + p1_r[...]
    z = (jnp.dot(a, wrel_r[...]) + jnp.dot(h1_r[...], wroot_r[...]) +
         b_r[0:1])
    h2_r[...] = _ln_lrelu(z, g_r[0:1], bb_r[0:1])

  return pl.pallas_call(
      body,
      grid=(N // RB,),
      in_specs=[
          _rows(64), _rows(64), _rows(64),
          _full((64, 128)), _full((64, 128)),
          _full((8, 128)), _full((8, 128)), _full((8, 128)),
      ],
      out_specs=[_rows(128)],
      out_shape=[jax.ShapeDtypeStruct((NPAD, 128), jnp.float32)],
  )(h1, p0, p1, wrel1, wroot1, _rep8(b1), _rep8(g1), _rep8(bb1))


def _k_layer2(h2, p0, p1, wrel2, wroot2, b2, g2, bb2):
  def body(h2_r, p0_r, p1_r, wrel_r, wroot_r, b_r, g_r, bb_r, o0_r, o1_r):
    a = p0_r[...] + p1_r[...]
    z = jnp.dot(a, wrel_r[...]) + jnp.dot(h2_r[...], wroot_r[...]) + b_r[0:1]
    h3 = _ln_lrelu(z, g_r[0:1], bb_r[0:1])
    o0_r[...] = h3[:, :128]
    o1_r[...] = h3[:, 128:]

  return pl.pallas_call(
      body,
      grid=(N // RB,),
      in_specs=[
          _rows(128), _rows(128), _rows(128),
          _full((128, 256)), _full((128, 256)),
          _full((8, 256)), _full((8, 256)), _full((8, 256)),
      ],
      out_specs=[_rows(128), _rows(128)],
      out_shape=[jax.ShapeDtypeStruct((NPAD, 128), jnp.float32)] * 2,
  )(h2, p0, p1, wrel2, wroot2, _rep8(b2), _rep8(g2), _rep8(bb2))


def _k_layer3(h3s, a3s, wrel3, wroot3, b3, g3, bb3):
  def body(h30_r, h31_r, a30_r, a31_r, wrel_r, wroot_r, b_r, g_r, bb_r,
           *outs):
    z = (jnp.dot(a30_r[...], wrel_r[0:128]) +
         jnp.dot(a31_r[...], wrel_r[128:256]) +
         jnp.dot(h30_r[...], wroot_r[0:128]) +
         jnp.dot(h31_r[...], wroot_r[128:256]) + b_r[0:1])
    h4 = _ln_lrelu(z, g_r[0:1], bb_r[0:1])
    for c in range(4):
      outs[c][...] = h4[:, c * 128:(c + 1) * 128]

  return pl.pallas_call(
      body,
      grid=(N // RB,),
      in_specs=[
          _rows(128), _rows(128), _rows(128), _rows(128),
          _full((256, 512)), _full((256, 512)),
          _full((8, 512)), _full((8, 512)), _full((8, 512)),
      ],
      out_specs=[_rows(128)] * 4,
      out_shape=[jax.ShapeDtypeStruct((NPAD, 128), jnp.float32)] * 4,
  )(*h3s, *a3s, wrel3, wroot3, _rep8(b3), _rep8(g3), _rep8(bb3))


def _k_final(h4s, a4s, wrel4, wroot4, b4, g4, bb4, wfc, bfc):
  nblocks = N // RB

  def body(h40_r, h41_r, h42_r, h43_r, a40_r, a41_r, a42_r, a43_r,
           wrel_r, wroot_r, b_r, g_r, bb_r, wfc_r, bfc_r, out_r, acc):
    i = pl.program_id(0)
    hs = (h40_r, h41_r, h42_r, h43_r)
    as_ = (a40_r, a41_r, a42_r, a43_r)
    z = b_r[0:1] + jnp.zeros((RB, 512), jnp.float32)
    for c in range(4):
      z = z + jnp.dot(as_[c][...], wrel_r[pl.ds(c * 128, 128)])
      z = z + jnp.dot(hs[c][...], wroot_r[pl.ds(c * 128, 128)])
    h5 = _ln_lrelu(z, g_r[0:1], bb_r[0:1])
    part = jnp.sum(h5.reshape(RB // 8, 8, 512), axis=0)

    @pl.when(i == 0)
    def _():
      acc[...] = part

    @pl.when(i > 0)
    def _():
      acc[...] = acc[...] + part

    @pl.when(i == nblocks - 1)
    def _():
      total = jnp.sum(acc[...] * wfc_r[0:1]) + bfc_r[0, 0]
      out_r[...] = jnp.full((8, 128), total, jnp.float32)

  return pl.pallas_call(
      body,
      grid=(nblocks,),
      in_specs=[_rows(128)] * 8 + [
          _full((512, 512)), _full((512, 512)),
          _full((8, 512)), _full((8, 512)), _full((8, 512)),
          _full((8, 512)), _full((8, 128)),
      ],
      out_specs=[pl.BlockSpec((8, 128), lambda i: (0, 0))],
      out_shape=[jax.ShapeDtypeStruct((8, 128), jnp.float32)],
      scratch_shapes=[pltpu.VMEM((8, 512), jnp.float32)],
  )(*h4s, *a4s, wrel4, wroot4, _rep8(b4), _rep8(g4), _rep8(bb4),
    _rep8(wfc.reshape(-1)), jnp.broadcast_to(bfc.reshape(1, 1), (8, 128)))


def kernel(x, edge_index, params):
  src = edge_index[0].astype(jnp.int32)
  dst = edge_index[1].astype(jnp.int32)
  npad_e = E_PAD - E
  pad_src = (jnp.arange(npad_e, dtype=jnp.int32) * 61) % N
  pad_dst = N + (jnp.arange(npad_e, dtype=jnp.int32) % (NPAD - N))
  src_p = jnp.concatenate([src, pad_src])
  dst_p = jnp.concatenate([dst, pad_dst])
  dst2d = dst_p.reshape(E_PAD // 128, 128)

  seg_edge64 = _make_segsum(1, edge_split=True, dc=64)
  seg_edge = _make_segsum(1, edge_split=True, dc=128)
  seg_feat2 = _make_segsum(2, edge_split=False, dc=128)
  seg_feat4 = _make_segsum(4, edge_split=False, dc=128)

  p = params
  y0, r0 = _k_pre(x, p["W_rel0"], p["W_root0"], p["b0"])
  q0, q1 = seg_edge64(y0, src_p, dst2d)
  h1 = _k_layer0(r0, q0, q1, p["ln_g0"], p["ln_b0"])[0]
  q0, q1 = seg_edge64(h1, src_p, dst2d)
  h2 = _k_layer1(h1, q0, q1, p["W_rel1"], p["W_root1"], p["b1"],
                 p["ln_g1"], p["ln_b1"])[0]
  q0, q1 = seg_edge(h2, src_p, dst2d)
  h3s = _k_layer2(h2, q0, q1, p["W_rel2"], p["W_root2"], p["b2"],
                  p["ln_g2"], p["ln_b2"])
  a3s = seg_feat2(*h3s, src_p, dst2d)
  h4s = _k_layer3(h3s, a3s, p["W_rel3"], p["W_root3"], p["b3"],
                  p["ln_g3"], p["ln_b3"])
  a4s = seg_feat4(*h4s, src_p, dst2d)
  out = _k_final(h4s, a4s, p["W_rel4"], p["W_root4"], p["b4"],
                 p["ln_g4"], p["ln_b4"], p["W_fc"], p["b_fc"])[0]
  return out[0:1, 0]
